# Initial kernel scaffold; baseline (speedup 1.0000x reference)
import jax
import jax.numpy as jnp
from jax.experimental import pallas as pl


def kernel(x, edge_index, edge_weight, W0a, b0a, W0b, b0b, W1a, b1a, W1b, b1b, W2a, b2a, W2b, b2b, Wl1, bl1, Wl2, bl2, eps0, gamma0, beta0, eps1, gamma1, beta1, eps2, gamma2, beta2):
    raise NotImplementedError("write your pallas kernel here")


# speedup vs baseline: 13.9964x; 13.9964x over previous
import functools

import jax
import jax.numpy as jnp
from jax import lax
from jax.experimental import pallas as pl
from jax.experimental.pallas import tpu as pltpu
from jax.experimental.pallas import tpu_sc as plsc

N = 10000
H = 16
NC, NS = 2, 16
NW = NC * NS
GROUP = 128
GROUPS_PER_W = 80
E_PAD = NW * GROUPS_PER_W * GROUP
N_PAD = 10240
ROWS_PER_TILE = N_PAD // NS
FIRE = 16


def _seg_sum_body(y_hbm, src_hbm, dst_hbm, zeros_hbm, out_hbm,
                  src_v, dst_v, rows_v, acc_sh, sem_g, sem_s):
    cid = lax.axis_index("c")
    sid = lax.axis_index("s")
    wid = cid * NS + sid

    row0 = sid * ROWS_PER_TILE
    pltpu.sync_copy(zeros_hbm.at[pl.ds(row0, ROWS_PER_TILE)],
                    acc_sh.at[pl.ds(row0, ROWS_PER_TILE)])
    g0 = wid * GROUPS_PER_W
    pltpu.sync_copy(src_hbm.at[pl.ds(g0, GROUPS_PER_W)], src_v)
    pltpu.sync_copy(dst_hbm.at[pl.ds(g0, GROUPS_PER_W)], dst_v)
    plsc.subcore_barrier()

    n_chunks = GROUPS_PER_W // FIRE

    def fire_gathers(chunk, s):
        for b in range(FIRE):
            pltpu.async_copy(
                y_hbm.at[src_v.at[chunk * FIRE + b]], rows_v.at[s, b], sem_g)

    def wait_gathers(s):
        for b in range(FIRE):
            pltpu.make_async_copy(
                y_hbm.at[src_v.at[b]], rows_v.at[s, b], sem_g).wait()

    def fire_scatters(chunk, s):
        for b in range(FIRE):
            pltpu.async_copy(
                rows_v.at[s, b], acc_sh.at[dst_v.at[chunk * FIRE + b]],
                sem_s, add=True)

    def wait_scatters(s):
        for b in range(FIRE):
            pltpu.make_async_copy(
                rows_v.at[s, b], acc_sh.at[dst_v.at[b]], sem_s).wait()

    fire_gathers(0, 0)
    for i in range(n_chunks):
        s = i % 2
        wait_gathers(s)
        fire_scatters(i, s)
        if i + 1 < n_chunks:
            if i >= 1:
                wait_scatters(1 - s)
            fire_gathers(i + 1, 1 - s)
    wait_scatters((n_chunks - 1) % 2)
    if n_chunks > 1:
        wait_scatters(n_chunks % 2)
    plsc.subcore_barrier()
    pltpu.sync_copy(acc_sh.at[pl.ds(row0, ROWS_PER_TILE)],
                    out_hbm.at[cid, pl.ds(row0, ROWS_PER_TILE)])


@jax.jit
def _seg_sum(y, src_g, dst_g, zeros):
    return pl.kernel(
        _seg_sum_body,
        out_type=jax.ShapeDtypeStruct((NC, N_PAD, H), jnp.float32),
        mesh=plsc.VectorSubcoreMesh(core_axis_name="c", subcore_axis_name="s"),
        scratch_types=[
            pltpu.VMEM((GROUPS_PER_W, GROUP), jnp.int32),
            pltpu.VMEM((GROUPS_PER_W, GROUP), jnp.int32),
            pltpu.VMEM((2, FIRE, GROUP, H), jnp.float32),
            pltpu.VMEM_SHARED((N_PAD, H), jnp.float32),
            pltpu.SemaphoreType.DMA,
            pltpu.SemaphoreType.DMA,
        ],
        compiler_params=pltpu.CompilerParams(use_tc_tiling_on_sc=False),
    )(y, src_g, dst_g, zeros)


def _proj_body(x_ref, w_ref, o_ref):
    o_ref[...] = jnp.dot(x_ref[...], w_ref[...],
                         preferred_element_type=jnp.float32)


@jax.jit
def _proj(x, w):
    return pl.pallas_call(
        _proj_body,
        out_shape=jax.ShapeDtypeStruct((N, H), jnp.float32),
    )(x, w)


def _bn(w, gamma, beta):
    mean = jnp.mean(w, axis=0, keepdims=True)
    var = jnp.mean(w * w, axis=0, keepdims=True) - mean * mean
    return (w - mean) * lax.rsqrt(var + 1e-5) * gamma + beta


def _combine_body(y_ref, p_ref, sc_ref, ba_ref, wb_ref, bb_ref,
                  g_ref, be_ref, wn_ref, o_ref):
    agg = p_ref[0, :N, :] + p_ref[1, :N, :]
    u = jnp.maximum(sc_ref[0, 0] * y_ref[...] + agg + ba_ref[...], 0.0)
    v = jnp.dot(u, wb_ref[...], preferred_element_type=jnp.float32)
    v = jnp.maximum(v + bb_ref[...], 0.0)
    s = _bn(v, g_ref[...], be_ref[...])
    o_ref[...] = jnp.dot(s, wn_ref[...], preferred_element_type=jnp.float32)


@jax.jit
def _combine(y, parts, scale, ba, Wb, bb, gamma, beta, Wnext):
    return pl.pallas_call(
        _combine_body,
        out_shape=jax.ShapeDtypeStruct((N, H), jnp.float32),
    )(y, parts, scale, ba, Wb, bb, gamma, beta, Wnext)


def _head_body(y_ref, p_ref, sc_ref, ba_ref, wb_ref, bb_ref,
               g_ref, be_ref, w1_ref, b1_ref, w2_ref, b2_ref, o_ref):
    agg = p_ref[0, :N, :] + p_ref[1, :N, :]
    u = jnp.maximum(sc_ref[0, 0] * y_ref[...] + agg + ba_ref[...], 0.0)
    v = jnp.dot(u, wb_ref[...], preferred_element_type=jnp.float32)
    v = jnp.maximum(v + bb_ref[...], 0.0)
    s = _bn(v, g_ref[...], be_ref[...])
    t = jnp.dot(s, w1_ref[...], preferred_element_type=jnp.float32)
    t = jnp.maximum(t + b1_ref[...], 0.0)
    o = jnp.dot(t, w2_ref[...], preferred_element_type=jnp.float32)
    o = o + b2_ref[...]
    m = jnp.max(o, axis=-1, keepdims=True)
    e = jnp.exp(o - m)
    o_ref[...] = (o - m) - jnp.log(jnp.sum(e, axis=-1, keepdims=True))


@jax.jit
def _head(y, parts, scale, ba, Wb, bb, gamma, beta, W1, b1, W2, b2):
    C = W2.shape[1]
    return pl.pallas_call(
        _head_body,
        out_shape=jax.ShapeDtypeStruct((N, C), jnp.float32),
    )(y, parts, scale, ba, Wb, bb, gamma, beta, W1, b1, W2, b2)


def kernel(x, edge_index, edge_weight, W0a, b0a, W0b, b0b, W1a, b1a, W1b,
           b1b, W2a, b2a, W2b, b2b, Wl1, bl1, Wl2, bl2, eps0, gamma0, beta0,
           eps1, gamma1, beta1, eps2, gamma2, beta2):
    E = edge_index.shape[1]
    src = edge_index[0].astype(jnp.int32)
    dst = edge_index[1].astype(jnp.int32)
    pad = E_PAD - E
    src_g = jnp.concatenate([src, jnp.zeros((pad,), jnp.int32)])
    src_g = src_g.reshape(NW * GROUPS_PER_W, GROUP)
    dst_g = jnp.concatenate([dst, jnp.full((pad,), N, jnp.int32)])
    dst_g = dst_g.reshape(NW * GROUPS_PER_W, GROUP)
    zeros = jnp.zeros((N_PAD, H), jnp.float32)

    row = lambda b: b.reshape(1, -1)
    sc0 = (1.0 + eps0).reshape(1, 1)
    sc1 = (1.0 + eps1).reshape(1, 1)
    sc2 = (1.0 + eps2).reshape(1, 1)

    y0 = _proj(x, W0a)
    p0 = _seg_sum(y0, src_g, dst_g, zeros)
    y1 = _combine(y0, p0, sc0, row(b0a), W0b, row(b0b),
                  row(gamma0), row(beta0), W1a)
    p1 = _seg_sum(y1, src_g, dst_g, zeros)
    y2 = _combine(y1, p1, sc1, row(b1a), W1b, row(b1b),
                  row(gamma1), row(beta1), W2a)
    p2 = _seg_sum(y2, src_g, dst_g, zeros)
    out = _head(y2, p2, sc2, row(b2a), W2b, row(b2b),
                row(gamma2), row(beta2), Wl1, row(bl1), Wl2, row(bl2))
    return out
